# Initial kernel scaffold; baseline (speedup 1.0000x reference)
import jax
import jax.numpy as jnp
from jax.experimental import pallas as pl


def kernel(main_category_id, sub_category_id, main_table, sub_table):
    raise NotImplementedError("write your pallas kernel here")


# speedup vs baseline: 6.9037x; 6.9037x over previous
import functools

import jax
import jax.numpy as jnp
from jax import lax
from jax.experimental import pallas as pl
from jax.experimental.pallas import tpu as pltpu
from jax.experimental.pallas import tpu_sc as plsc

MAIN_DIM = 16
SUB_DIM = 48
OUT_DIM = MAIN_DIM + SUB_DIM

NUM_CORES = 2
NUM_SUBCORES = 16
NUM_WORKERS = NUM_CORES * NUM_SUBCORES

CHUNK = 1024
IDX_W = 128
KB = CHUNK // IDX_W


def _encoder(n_rows):
    per_w = n_rows // NUM_WORKERS
    n_chunks = per_w // CHUNK
    mesh = plsc.VectorSubcoreMesh(core_axis_name="c", subcore_axis_name="s")

    @functools.partial(
        pl.kernel,
        mesh=mesh,
        compiler_params=pltpu.CompilerParams(use_tc_tiling_on_sc=False),
        out_type=jax.ShapeDtypeStruct((n_rows, OUT_DIM), jnp.float32),
        scratch_types=[
            pltpu.VMEM((KB, IDX_W), jnp.int32),
            pltpu.VMEM((KB, IDX_W), jnp.int32),
            pltpu.VMEM((CHUNK, MAIN_DIM), jnp.float32),
            pltpu.VMEM((CHUNK, SUB_DIM), jnp.float32),
            pltpu.SemaphoreType.DMA,
            pltpu.SemaphoreType.DMA,
        ],
    )
    def enc(mid_hbm, sid_hbm, mtab_hbm, stab_hbm, out_hbm,
            idx_m, idx_s, mbuf, sbuf, sem_m, sem_s):
        wid = lax.axis_index("s") * NUM_CORES + lax.axis_index("c")

        def body(t, carry):
            base = pl.multiple_of(wid * per_w + t * CHUNK, CHUNK)
            rb = pl.multiple_of(base // IDX_W, KB)
            pltpu.sync_copy(mid_hbm.at[pl.ds(rb, KB)], idx_m)
            pltpu.sync_copy(sid_hbm.at[pl.ds(rb, KB)], idx_s)
            waits = []
            for j in range(KB):
                waits.append(pltpu.async_copy(
                    mtab_hbm.at[idx_m.at[j]],
                    mbuf.at[pl.ds(j * IDX_W, IDX_W)], sem_m))
                waits.append(pltpu.async_copy(
                    stab_hbm.at[idx_s.at[j]],
                    sbuf.at[pl.ds(j * IDX_W, IDX_W)], sem_s))
            for w in waits:
                w.wait()
            pltpu.sync_copy(mbuf, out_hbm.at[pl.ds(base, CHUNK),
                                             pl.ds(0, MAIN_DIM)])
            pltpu.sync_copy(sbuf, out_hbm.at[pl.ds(base, CHUNK),
                                             pl.ds(MAIN_DIM, SUB_DIM)])
            return carry

        lax.fori_loop(0, n_chunks, body, 0)

    return enc


def kernel(main_category_id, sub_category_id, main_table, sub_table):
    b, h = main_category_id.shape
    n = b * h
    mid = main_category_id.reshape(n // IDX_W, IDX_W).astype(jnp.int32)
    sid = sub_category_id.reshape(n // IDX_W, IDX_W).astype(jnp.int32)
    out = _encoder(n)(mid, sid, main_table, sub_table)
    return out.reshape(b, h, OUT_DIM)


# speedup vs baseline: 7.1061x; 1.0293x over previous
import functools

import jax
import jax.numpy as jnp
from jax import lax
from jax.experimental import pallas as pl
from jax.experimental.pallas import tpu as pltpu
from jax.experimental.pallas import tpu_sc as plsc

MAIN_DIM = 16
SUB_DIM = 48
OUT_DIM = MAIN_DIM + SUB_DIM

NUM_CORES = 2
NUM_SUBCORES = 16
NUM_WORKERS = NUM_CORES * NUM_SUBCORES

CHUNK = 512
IDX_W = 128
KB = CHUNK // IDX_W
NBUF = 2


def _encoder(n_rows):
    per_w = n_rows // NUM_WORKERS
    n_chunks = per_w // CHUNK
    assert per_w % CHUNK == 0 and n_chunks % 2 == 0 and n_chunks >= 4
    mesh = plsc.VectorSubcoreMesh(core_axis_name="c", subcore_axis_name="s")

    @functools.partial(
        pl.kernel,
        mesh=mesh,
        compiler_params=pltpu.CompilerParams(use_tc_tiling_on_sc=False),
        out_type=jax.ShapeDtypeStruct((n_rows, OUT_DIM), jnp.float32),
        scratch_types=[
            pltpu.VMEM((NBUF * KB, IDX_W), jnp.int32),
            pltpu.VMEM((NBUF * KB, IDX_W), jnp.int32),
            pltpu.VMEM((NBUF * CHUNK, MAIN_DIM), jnp.float32),
            pltpu.VMEM((NBUF * CHUNK, SUB_DIM), jnp.float32),
            pltpu.SemaphoreType.DMA,
            pltpu.SemaphoreType.DMA,
            pltpu.SemaphoreType.DMA,
            pltpu.SemaphoreType.DMA,
            pltpu.SemaphoreType.DMA,
            pltpu.SemaphoreType.DMA,
        ],
    )
    def enc(mid_hbm, sid_hbm, mtab_hbm, stab_hbm, out_hbm,
            idx_m, idx_s, mbuf, sbuf,
            sem_gm0, sem_gm1, sem_gs0, sem_gs1, sem_w0, sem_w1):
        wid = lax.axis_index("s") * NUM_CORES + lax.axis_index("c")
        sem_gm = (sem_gm0, sem_gm1)
        sem_gs = (sem_gs0, sem_gs1)
        sem_w = (sem_w0, sem_w1)

        def row_base(t):
            return pl.multiple_of(wid * per_w + t * CHUNK, CHUNK)

        def fire(t, b):
            rb = row_base(t) // IDX_W
            pltpu.sync_copy(mid_hbm.at[pl.ds(rb, KB)],
                            idx_m.at[pl.ds(b * KB, KB)])
            pltpu.sync_copy(sid_hbm.at[pl.ds(rb, KB)],
                            idx_s.at[pl.ds(b * KB, KB)])
            for j in range(KB):
                pltpu.async_copy(
                    mtab_hbm.at[idx_m.at[b * KB + j]],
                    mbuf.at[pl.ds(b * CHUNK + j * IDX_W, IDX_W)], sem_gm[b])
                pltpu.async_copy(
                    stab_hbm.at[idx_s.at[b * KB + j]],
                    sbuf.at[pl.ds(b * CHUNK + j * IDX_W, IDX_W)], sem_gs[b])

        def drain_gathers(b):
            for j in range(KB):
                pltpu.make_async_copy(
                    mtab_hbm.at[idx_m.at[b * KB + j]],
                    mbuf.at[pl.ds(b * CHUNK + j * IDX_W, IDX_W)],
                    sem_gm[b]).wait()
                pltpu.make_async_copy(
                    stab_hbm.at[idx_s.at[b * KB + j]],
                    sbuf.at[pl.ds(b * CHUNK + j * IDX_W, IDX_W)],
                    sem_gs[b]).wait()

        def write(t, b):
            base = row_base(t)
            pltpu.async_copy(mbuf.at[pl.ds(b * CHUNK, CHUNK)],
                             out_hbm.at[pl.ds(base, CHUNK),
                                        pl.ds(0, MAIN_DIM)], sem_w[b])
            pltpu.async_copy(sbuf.at[pl.ds(b * CHUNK, CHUNK)],
                             out_hbm.at[pl.ds(base, CHUNK),
                                        pl.ds(MAIN_DIM, SUB_DIM)], sem_w[b])

        def drain_write(t, b):
            base = row_base(t)
            pltpu.make_async_copy(mbuf.at[pl.ds(b * CHUNK, CHUNK)],
                                  out_hbm.at[pl.ds(base, CHUNK),
                                             pl.ds(0, MAIN_DIM)],
                                  sem_w[b]).wait()
            pltpu.make_async_copy(sbuf.at[pl.ds(b * CHUNK, CHUNK)],
                                  out_hbm.at[pl.ds(base, CHUNK),
                                             pl.ds(MAIN_DIM, SUB_DIM)],
                                  sem_w[b]).wait()

        fire(0, 0)
        drain_gathers(0)
        fire(1, 1)
        write(0, 0)

        def pair(i, carry):
            t0 = 2 * i + 1
            drain_gathers(1)
            drain_write(t0 - 1, 0)
            fire(t0 + 1, 0)
            write(t0, 1)
            t1 = 2 * i + 2
            drain_gathers(0)
            drain_write(t1 - 1, 1)
            fire(t1 + 1, 1)
            write(t1, 0)
            return carry

        lax.fori_loop(0, (n_chunks - 2) // 2, pair, 0)

        t_last = n_chunks - 1
        drain_gathers(1)
        drain_write(t_last - 1, 0)
        write(t_last, 1)
        drain_write(t_last, 1)

    return enc


def kernel(main_category_id, sub_category_id, main_table, sub_table):
    b, h = main_category_id.shape
    n = b * h
    mid = main_category_id.reshape(n // IDX_W, IDX_W).astype(jnp.int32)
    sid = sub_category_id.reshape(n // IDX_W, IDX_W).astype(jnp.int32)
    out = _encoder(n)(mid, sid, main_table, sub_table)
    return out.reshape(b, h, OUT_DIM)
